# baseline (speedup 1.0000x reference)
import functools

import jax
import jax.numpy as jnp
from jax import lax
from jax.experimental import pallas as pl
from jax.experimental.pallas import tpu as pltpu
from jax.experimental.pallas import tpu_sc as plsc

BATCH = 16384
K = 16
NC = 2
NS = 16
L = 16
NW = NC * NS
BPW = BATCH // NW
NGRP = BPW // L
SLAB = 128

_mesh = plsc.VectorSubcoreMesh(core_axis_name="c", subcore_axis_name="s")


@functools.partial(
    pl.kernel,
    out_type=jax.ShapeDtypeStruct((BATCH,), jnp.float32),
    mesh=_mesh,
    compiler_params=pltpu.CompilerParams(
        needs_layout_passes=False,
        disable_bounds_checks=True,
    ),
)
def _mf_predict(uidx_hbm, iidx_hbm, wt_hbm, ht_hbm, out_hbm):
    def body(uidx, iidx, uslab0, uslab1, vslab, outv, semu0, semu1, semv):
        wid = lax.axis_index("s") * NC + lax.axis_index("c")
        base = wid * BPW
        iota = lax.iota(jnp.int32, L)

        pltpu.sync_copy(uidx_hbm.at[pl.ds(base, BPW)], uidx)
        pltpu.sync_copy(iidx_hbm.at[pl.ds(base, BPW)], iidx)

        def fire_u(g, buf, sem):
            gc = jnp.minimum(g, NGRP - 1)
            ustart = (uidx[pl.ds(gc * L, L)] >> 7) << 7
            for j in range(L):
                us = pl.multiple_of(ustart[j], SLAB)
                pltpu.async_copy(
                    wt_hbm.at[:, pl.ds(us, SLAB)], buf.at[j], sem)

        def drain_u(buf, sem):
            for j in range(L):
                pltpu.make_async_copy(
                    wt_hbm.at[:, pl.ds(0, SLAB)], buf.at[j], sem).wait()

        def half(g, buf, sem_self, buf_other, sem_other):
            uvec = uidx[pl.ds(g * L, L)]
            ivec = iidx[pl.ds(g * L, L)]
            istart = (ivec >> 7) << 7
            ulane = uvec & 127
            ilane = ivec & 127
            vcopies = []
            for j in range(L):
                hs = pl.multiple_of(istart[j], SLAB)
                vcopies.append(pltpu.async_copy(
                    ht_hbm.at[:, pl.ds(hs, SLAB)], vslab.at[j], semv))
            drain_u(buf, sem_self)
            for cp in vcopies:
                cp.wait()
            acc = jnp.zeros((L,), jnp.float32)
            for d in range(K):
                dsplat = jnp.full((L,), d, jnp.int32)
                u = plsc.load_gather(buf, [iota, dsplat, ulane])
                v = plsc.load_gather(vslab, [iota, dsplat, ilane])
                acc = acc + u * v
            sig = 1.0 / (1.0 + jnp.exp(-acc))
            outv[pl.ds(g * L, L)] = sig
            fire_u(g + 2, buf, sem_self)

        fire_u(jnp.int32(0), uslab0, semu0)
        fire_u(jnp.int32(1), uslab1, semu1)

        def pair_body(i, carry):
            half(2 * i, uslab0, semu0, uslab1, semu1)
            half(2 * i + 1, uslab1, semu1, uslab0, semu0)
            return carry

        lax.fori_loop(0, NGRP // 2, pair_body, 0)

        drain_u(uslab0, semu0)
        drain_u(uslab1, semu1)

        pltpu.sync_copy(outv, out_hbm.at[pl.ds(base, BPW)])

    pl.run_scoped(
        body,
        pltpu.VMEM((BPW,), jnp.int32),
        pltpu.VMEM((BPW,), jnp.int32),
        pltpu.VMEM((L, K, SLAB), jnp.float32),
        pltpu.VMEM((L, K, SLAB), jnp.float32),
        pltpu.VMEM((L, K, SLAB), jnp.float32),
        pltpu.VMEM((BPW,), jnp.float32),
        pltpu.SemaphoreType.DMA,
        pltpu.SemaphoreType.DMA,
        pltpu.SemaphoreType.DMA,
    )


def kernel(x, W, H):
    return _mf_predict(x[:, 0], x[:, 1], W.T, H.T)
